# baseline (speedup 1.0000x reference)
import jax
import jax.numpy as jnp
from jax import lax
from jax.experimental import pallas as pl
from jax.experimental.pallas import tpu as pltpu
from jax.experimental.pallas import tpu_sc as plsc

NC = 2
NS = 16
L = 16
NW = NC * NS


def _sc_body(bpw, dim, vi_hbm, vj_hbm, ft_hbm, st_hbm, ct_hbm,
             out1_hbm, out2_hbm,
             idx_i, idx_j, ra, rb, rc, rd, o1, o2, sem):
    wid = lax.axis_index("s") * NC + lax.axis_index("c")
    base = wid * bpw
    pltpu.sync_copy(vi_hbm.at[pl.ds(base, bpw)], idx_i)
    pltpu.sync_copy(vj_hbm.at[pl.ds(base, bpw)], idx_j)

    def fire(g, carry):
        gsl = pl.ds(g * L, L)
        ii = idx_i[gsl]
        jj = idx_j[gsl]
        for d in range(dim):
            pltpu.async_copy(ft_hbm.at[d].at[ii], ra.at[d].at[gsl], sem)
            pltpu.async_copy(ft_hbm.at[d].at[jj], rb.at[d].at[gsl], sem)
            pltpu.async_copy(st_hbm.at[d].at[ii], rc.at[d].at[gsl], sem)
            pltpu.async_copy(ct_hbm.at[d].at[jj], rd.at[d].at[gsl], sem)
        return carry

    lax.fori_loop(0, bpw // L, fire, 0)

    def drain(g, carry):
        gsl = pl.ds(g * L, L)
        for d in range(dim):
            pltpu.make_async_copy(ft_hbm.at[d].at[pl.ds(0, L)],
                                  ra.at[d].at[gsl], sem).wait()
            pltpu.make_async_copy(ft_hbm.at[d].at[pl.ds(0, L)],
                                  rb.at[d].at[gsl], sem).wait()
            pltpu.make_async_copy(st_hbm.at[d].at[pl.ds(0, L)],
                                  rc.at[d].at[gsl], sem).wait()
            pltpu.make_async_copy(ct_hbm.at[d].at[pl.ds(0, L)],
                                  rd.at[d].at[gsl], sem).wait()
        return carry

    lax.fori_loop(0, bpw // L, drain, 0)

    def group(g, carry):
        gsl = pl.ds(g * L, L)
        acc1 = jnp.zeros((L,), jnp.float32)
        acc2 = jnp.zeros((L,), jnp.float32)
        for d in range(dim):
            acc1 = acc1 + ra[d, gsl] * rb[d, gsl]
            acc2 = acc2 + rc[d, gsl] * rd[d, gsl]
        o1[gsl] = acc1
        o2[gsl] = acc2
        return carry

    lax.fori_loop(0, bpw // L, group, 0)
    pltpu.sync_copy(o1, out1_hbm.at[pl.ds(base, bpw)])
    pltpu.sync_copy(o2, out2_hbm.at[pl.ds(base, bpw)])


def kernel(v_i, v_j, first_table, second_table, context_table):
    batch = v_i.shape[0]
    nodes, dim = first_table.shape
    assert batch % (NW * L) == 0 and dim == L
    bpw = batch // NW
    v_i = v_i.astype(jnp.int32)
    v_j = v_j.astype(jnp.int32)
    ftt = first_table.T
    stt = second_table.T
    ctt = context_table.T

    mesh = plsc.VectorSubcoreMesh(core_axis_name="c", subcore_axis_name="s")
    f = pl.kernel(
        lambda *refs: _sc_body(bpw, dim, *refs),
        out_type=(
            jax.ShapeDtypeStruct((batch,), jnp.float32),
            jax.ShapeDtypeStruct((batch,), jnp.float32),
        ),
        mesh=mesh,
        compiler_params=pltpu.CompilerParams(
            needs_layout_passes=False, use_tc_tiling_on_sc=False
        ),
        scratch_types=[
            pltpu.VMEM((bpw,), jnp.int32),
            pltpu.VMEM((bpw,), jnp.int32),
            pltpu.VMEM((dim, bpw), jnp.float32),
            pltpu.VMEM((dim, bpw), jnp.float32),
            pltpu.VMEM((dim, bpw), jnp.float32),
            pltpu.VMEM((dim, bpw), jnp.float32),
            pltpu.VMEM((bpw,), jnp.float32),
            pltpu.VMEM((bpw,), jnp.float32),
            pltpu.SemaphoreType.DMA,
        ],
    )
    first, second = f(v_i, v_j, ftt, stt, ctt)
    return (first, second)


# speedup vs baseline: 21.0869x; 21.0869x over previous
import jax
import jax.numpy as jnp
from jax import lax
from jax.experimental import pallas as pl
from jax.experimental.pallas import tpu as pltpu
from jax.experimental.pallas import tpu_sc as plsc

NC = 2
NS = 16
L = 16
NW = NC * NS


def _sc_body(bpw, dim, vi_hbm, vj_hbm, ft_hbm, st_hbm, ct_hbm,
             out1_hbm, out2_hbm,
             idx_i, idx_j, ra, rb, rc, rd, o1, o2, sem):
    wid = lax.axis_index("s") * NC + lax.axis_index("c")
    base = wid * bpw
    pltpu.sync_copy(vi_hbm.at[pl.ds(base, bpw)], idx_i)
    pltpu.sync_copy(vj_hbm.at[pl.ds(base, bpw)], idx_j)

    def fire(g, carry):
        gsl = pl.ds(g * L, L)
        ii = idx_i[gsl]
        jj = idx_j[gsl]
        qi = jnp.left_shift(jnp.right_shift(ii, 7), 10) + jnp.bitwise_and(ii, 127)
        qj = jnp.left_shift(jnp.right_shift(jj, 7), 10) + jnp.bitwise_and(jj, 127)
        for d in range(dim):
            h = d >> 3
            off = (d & 7) * 128
            pltpu.async_copy(ft_hbm.at[h].at[qi + off], ra.at[d].at[gsl], sem)
            pltpu.async_copy(ft_hbm.at[h].at[qj + off], rb.at[d].at[gsl], sem)
            pltpu.async_copy(st_hbm.at[h].at[qi + off], rc.at[d].at[gsl], sem)
            pltpu.async_copy(ct_hbm.at[h].at[qj + off], rd.at[d].at[gsl], sem)
        return carry

    lax.fori_loop(0, bpw // L, fire, 0)

    def drain(g, carry):
        gsl = pl.ds(g * L, L)
        for d in range(dim):
            pltpu.make_async_copy(ft_hbm.at[0].at[pl.ds(0, L)],
                                  ra.at[d].at[gsl], sem).wait()
            pltpu.make_async_copy(ft_hbm.at[0].at[pl.ds(0, L)],
                                  rb.at[d].at[gsl], sem).wait()
            pltpu.make_async_copy(st_hbm.at[0].at[pl.ds(0, L)],
                                  rc.at[d].at[gsl], sem).wait()
            pltpu.make_async_copy(ct_hbm.at[0].at[pl.ds(0, L)],
                                  rd.at[d].at[gsl], sem).wait()
        return carry

    lax.fori_loop(0, bpw // L, drain, 0)

    def group(g, carry):
        gsl = pl.ds(g * L, L)
        acc1 = jnp.zeros((L,), jnp.float32)
        acc2 = jnp.zeros((L,), jnp.float32)
        for d in range(dim):
            acc1 = acc1 + ra[d, gsl] * rb[d, gsl]
            acc2 = acc2 + rc[d, gsl] * rd[d, gsl]
        o1[gsl] = acc1
        o2[gsl] = acc2
        return carry

    lax.fori_loop(0, bpw // L, group, 0)
    pltpu.sync_copy(o1, out1_hbm.at[pl.ds(base, bpw)])
    pltpu.sync_copy(o2, out2_hbm.at[pl.ds(base, bpw)])


def kernel(v_i, v_j, first_table, second_table, context_table):
    batch = v_i.shape[0]
    nodes, dim = first_table.shape
    assert batch % (NW * L) == 0 and dim == L
    bpw = batch // NW
    v_i = v_i.astype(jnp.int32)
    v_j = v_j.astype(jnp.int32)

    nwin = (nodes + 127) // 128
    npad = nwin * 128 - nodes

    def _native(t):
        tp = jnp.pad(t.T, ((0, 0), (0, npad)))
        x = tp.reshape(2, dim // 2, nwin, 128).transpose(0, 2, 1, 3)
        return x.reshape(2, nwin * (dim // 2) * 128)

    ftt = _native(first_table)
    stt = _native(second_table)
    ctt = _native(context_table)

    mesh = plsc.VectorSubcoreMesh(core_axis_name="c", subcore_axis_name="s")
    f = pl.kernel(
        lambda *refs: _sc_body(bpw, dim, *refs),
        out_type=(
            jax.ShapeDtypeStruct((batch,), jnp.float32),
            jax.ShapeDtypeStruct((batch,), jnp.float32),
        ),
        mesh=mesh,
        compiler_params=pltpu.CompilerParams(
            needs_layout_passes=False, use_tc_tiling_on_sc=False
        ),
        scratch_types=[
            pltpu.VMEM((bpw,), jnp.int32),
            pltpu.VMEM((bpw,), jnp.int32),
            pltpu.VMEM((dim, bpw), jnp.float32),
            pltpu.VMEM((dim, bpw), jnp.float32),
            pltpu.VMEM((dim, bpw), jnp.float32),
            pltpu.VMEM((dim, bpw), jnp.float32),
            pltpu.VMEM((bpw,), jnp.float32),
            pltpu.VMEM((bpw,), jnp.float32),
            pltpu.SemaphoreType.DMA,
        ],
    )
    first, second = f(v_i, v_j, ftt, stt, ctt)
    return (first, second)


# speedup vs baseline: 21.1099x; 1.0011x over previous
import jax
import jax.numpy as jnp
from jax import lax
from jax.experimental import pallas as pl
from jax.experimental.pallas import tpu as pltpu
from jax.experimental.pallas import tpu_sc as plsc

NC = 2
NS = 16
L = 16
NW = NC * NS


def _pair_body(bpw, dim, vi_hbm, vj_hbm, ta_hbm, tb_hbm, out_hbm,
               idx_i, idx_j, ra, rb, o1, sem):
    wid = lax.axis_index("s") * NC + lax.axis_index("c")
    base = wid * bpw
    pltpu.sync_copy(vi_hbm.at[pl.ds(base, bpw)], idx_i)
    pltpu.sync_copy(vj_hbm.at[pl.ds(base, bpw)], idx_j)

    def fire(g, carry):
        gsl = pl.ds(g * L, L)
        ii = idx_i[gsl]
        jj = idx_j[gsl]
        qi = jnp.left_shift(jnp.right_shift(ii, 7), 10) + jnp.bitwise_and(ii, 127)
        qj = jnp.left_shift(jnp.right_shift(jj, 7), 10) + jnp.bitwise_and(jj, 127)
        for d in range(dim):
            h = d >> 3
            off = (d & 7) * 128
            pltpu.async_copy(ta_hbm.at[h].at[qi + off], ra.at[d].at[gsl], sem)
            pltpu.async_copy(tb_hbm.at[h].at[qj + off], rb.at[d].at[gsl], sem)
        return carry

    lax.fori_loop(0, bpw // L, fire, 0)

    def drain(g, carry):
        gsl = pl.ds(g * L, L)
        for d in range(dim):
            pltpu.make_async_copy(ta_hbm.at[0].at[pl.ds(0, L)],
                                  ra.at[d].at[gsl], sem).wait()
            pltpu.make_async_copy(ta_hbm.at[0].at[pl.ds(0, L)],
                                  rb.at[d].at[gsl], sem).wait()
        return carry

    lax.fori_loop(0, bpw // L, drain, 0)

    def group(g, carry):
        gsl = pl.ds(g * L, L)
        acc = jnp.zeros((L,), jnp.float32)
        for d in range(dim):
            acc = acc + ra[d, gsl] * rb[d, gsl]
        o1[gsl] = acc
        return carry

    lax.fori_loop(0, bpw // L, group, 0)
    pltpu.sync_copy(o1, out_hbm.at[pl.ds(base, bpw)])


def _make_pair(batch, bpw, dim, half_words):
    mesh = plsc.VectorSubcoreMesh(core_axis_name="c", subcore_axis_name="s")
    return pl.kernel(
        lambda *refs: _pair_body(bpw, dim, *refs),
        out_type=jax.ShapeDtypeStruct((batch,), jnp.float32),
        mesh=mesh,
        compiler_params=pltpu.CompilerParams(
            needs_layout_passes=False, use_tc_tiling_on_sc=False
        ),
        scratch_types=[
            pltpu.VMEM((bpw,), jnp.int32),
            pltpu.VMEM((bpw,), jnp.int32),
            pltpu.VMEM((dim, bpw), jnp.float32),
            pltpu.VMEM((dim, bpw), jnp.float32),
            pltpu.VMEM((bpw,), jnp.float32),
            pltpu.SemaphoreType.DMA,
        ],
    )


def kernel(v_i, v_j, first_table, second_table, context_table):
    batch = v_i.shape[0]
    nodes, dim = first_table.shape
    assert batch % (NW * L) == 0 and dim == L
    bpw = batch // NW
    v_i = v_i.astype(jnp.int32)
    v_j = v_j.astype(jnp.int32)

    nwin = (nodes + 127) // 128
    npad = nwin * 128 - nodes
    half_words = nwin * (dim // 2) * 128

    def _native(t):
        tp = jnp.pad(t.T, ((0, 0), (0, npad)))
        x = tp.reshape(2, dim // 2, nwin, 128).transpose(0, 2, 1, 3)
        return x.reshape(2, half_words)

    pair = _make_pair(batch, bpw, dim, half_words)

    ftt = _native(first_table)
    first = pair(v_i, v_j, ftt, ftt)
    stt = _native(second_table)
    ctt = _native(context_table)
    second = pair(v_i, v_j, stt, ctt)
    return (first, second)


# speedup vs baseline: 21.1231x; 1.0006x over previous
import jax
import jax.numpy as jnp
from jax import lax
from jax.experimental import pallas as pl
from jax.experimental.pallas import tpu as pltpu
from jax.experimental.pallas import tpu_sc as plsc

NC = 2
NS = 16
L = 16
NW = NC * NS


def _sc_body(bpw, dim, vi_hbm, vj_hbm, ft_hbm, st_hbm, ct_hbm,
             out1_hbm, out2_hbm,
             idx_i, idx_j, ra, rb, rc, rd, o1, o2, sem):
    wid = lax.axis_index("s") * NC + lax.axis_index("c")
    base = wid * bpw
    pltpu.sync_copy(vi_hbm.at[pl.ds(base, bpw)], idx_i)
    pltpu.sync_copy(vj_hbm.at[pl.ds(base, bpw)], idx_j)

    def fire(g, carry):
        gsl = pl.ds(g * L, L)
        ii = idx_i[gsl]
        jj = idx_j[gsl]
        qi = jnp.left_shift(jnp.right_shift(ii, 7), 10) + jnp.bitwise_and(ii, 127)
        qj = jnp.left_shift(jnp.right_shift(jj, 7), 10) + jnp.bitwise_and(jj, 127)
        for d in range(dim):
            h = d >> 3
            off = (d & 7) * 128
            pltpu.async_copy(ft_hbm.at[h].at[qi + off], ra.at[d].at[gsl], sem)
            pltpu.async_copy(ft_hbm.at[h].at[qj + off], rb.at[d].at[gsl], sem)
            pltpu.async_copy(st_hbm.at[h].at[qi + off], rc.at[d].at[gsl], sem)
            pltpu.async_copy(ct_hbm.at[h].at[qj + off], rd.at[d].at[gsl], sem)
        return carry

    lax.fori_loop(0, bpw // L, fire, 0)

    def drain(g, carry):
        gsl = pl.ds(g * L, L)
        for d in range(dim):
            pltpu.make_async_copy(ft_hbm.at[0].at[pl.ds(0, L)],
                                  ra.at[d].at[gsl], sem).wait()
            pltpu.make_async_copy(ft_hbm.at[0].at[pl.ds(0, L)],
                                  rb.at[d].at[gsl], sem).wait()
            pltpu.make_async_copy(st_hbm.at[0].at[pl.ds(0, L)],
                                  rc.at[d].at[gsl], sem).wait()
            pltpu.make_async_copy(ct_hbm.at[0].at[pl.ds(0, L)],
                                  rd.at[d].at[gsl], sem).wait()
        return carry

    lax.fori_loop(0, bpw // L, drain, 0)

    def group(g, carry):
        gsl = pl.ds(g * L, L)
        acc1 = jnp.zeros((L,), jnp.float32)
        acc2 = jnp.zeros((L,), jnp.float32)
        for d in range(dim):
            acc1 = acc1 + ra[d, gsl] * rb[d, gsl]
            acc2 = acc2 + rc[d, gsl] * rd[d, gsl]
        o1[gsl] = acc1
        o2[gsl] = acc2
        return carry

    lax.fori_loop(0, bpw // L, group, 0)
    pltpu.sync_copy(o1, out1_hbm.at[pl.ds(base, bpw)])
    pltpu.sync_copy(o2, out2_hbm.at[pl.ds(base, bpw)])


def kernel(v_i, v_j, first_table, second_table, context_table):
    batch = v_i.shape[0]
    nodes, dim = first_table.shape
    assert batch % (NW * L) == 0 and dim == L
    bpw = batch // NW
    v_i = v_i.astype(jnp.int32)
    v_j = v_j.astype(jnp.int32)

    nwin = (nodes + 127) // 128
    npad = nwin * 128 - nodes

    def _native(t):
        tp = jnp.pad(t.T, ((0, 0), (0, npad)))
        x = tp.reshape(2, dim // 2, nwin, 128).transpose(0, 2, 1, 3)
        return x.reshape(2, nwin * (dim // 2) * 128)

    ftt = _native(first_table)
    stt = _native(second_table)
    ctt = _native(context_table)

    mesh = plsc.VectorSubcoreMesh(core_axis_name="c", subcore_axis_name="s")
    f = pl.kernel(
        lambda *refs: _sc_body(bpw, dim, *refs),
        out_type=(
            jax.ShapeDtypeStruct((batch,), jnp.float32),
            jax.ShapeDtypeStruct((batch,), jnp.float32),
        ),
        mesh=mesh,
        compiler_params=pltpu.CompilerParams(
            needs_layout_passes=False, use_tc_tiling_on_sc=False
        ),
        scratch_types=[
            pltpu.VMEM((bpw,), jnp.int32),
            pltpu.VMEM((bpw,), jnp.int32),
            pltpu.VMEM((dim, bpw), jnp.float32),
            pltpu.VMEM((dim, bpw), jnp.float32),
            pltpu.VMEM((dim, bpw), jnp.float32),
            pltpu.VMEM((dim, bpw), jnp.float32),
            pltpu.VMEM((bpw,), jnp.float32),
            pltpu.VMEM((bpw,), jnp.float32),
            pltpu.SemaphoreType.DMA,
        ],
    )
    first, second = f(v_i, v_j, ftt, stt, ctt)
    return (first, second)
